# baseline (speedup 1.0000x reference)
import functools

import jax
import jax.numpy as jnp
from jax import lax
from jax.experimental import pallas as pl
from jax.experimental.pallas import tpu as pltpu
from jax.experimental.pallas import tpu_sc as plsc

EMBED = 64
SEQ = 50
NUM_WORKERS = 32
BATCH = 4096
FLAT = BATCH * SEQ
PER_W = FLAT // NUM_WORKERS
CHUNK = 200
GATHER = 100
NCHUNK = PER_W // CHUNK
LANES = 16


def _build_kernel():
    mesh = plsc.VectorSubcoreMesh(core_axis_name="c", subcore_axis_name="s")

    @functools.partial(
        pl.kernel,
        mesh=mesh,
        compiler_params=pltpu.CompilerParams(use_tc_tiling_on_sc=False),
        out_type=jax.ShapeDtypeStruct((FLAT, EMBED), jnp.float32),
        scratch_types=[
            pltpu.VMEM((CHUNK // GATHER, GATHER), jnp.int32),
            pltpu.VMEM((CHUNK, EMBED), jnp.float32),
            pltpu.VMEM((SEQ, EMBED), jnp.float32),
            pltpu.SemaphoreType.DMA,
        ],
    )
    def k(x_hbm, table_hbm, pe_hbm, out_hbm, idx_v, rows_v, pe_v, sem):
        cid = lax.axis_index("c")
        sid = lax.axis_index("s")
        wid = sid * 2 + cid
        pltpu.sync_copy(pe_hbm, pe_v)

        def chunk_body(c, carry):
            r0 = wid * (PER_W // GATHER) + c * (CHUNK // GATHER)
            out0 = wid * PER_W + c * CHUNK
            pltpu.sync_copy(x_hbm.at[pl.ds(r0, CHUNK // GATHER)], idx_v)
            cps = [
                pltpu.async_copy(
                    table_hbm.at[idx_v.at[g]],
                    rows_v.at[pl.ds(g * GATHER, GATHER)],
                    sem,
                )
                for g in range(CHUNK // GATHER)
            ]
            for cp in cps:
                cp.wait()

            def add_elem(e, inner):
                for t in range(SEQ):
                    row = e * SEQ + t
                    for j in range(EMBED // LANES):
                        sl = pl.ds(j * LANES, LANES)
                        rows_v[row, sl] = rows_v[row, sl] + pe_v[t, sl]
                return inner

            lax.fori_loop(0, CHUNK // SEQ, add_elem, 0)
            pltpu.sync_copy(rows_v, out_hbm.at[pl.ds(out0, CHUNK)])
            return carry

        lax.fori_loop(0, NCHUNK, chunk_body, 0)

    return k


_sc_kernel = _build_kernel()


@jax.jit
def kernel(x, table, pe):
    x2 = x.reshape(FLAT // GATHER, GATHER).astype(jnp.int32)
    out = _sc_kernel(x2, table, pe)
    return out.reshape(BATCH, SEQ, EMBED)


# speedup vs baseline: 1.1971x; 1.1971x over previous
import functools

import jax
import jax.numpy as jnp
from jax import lax
from jax.experimental import pallas as pl
from jax.experimental.pallas import tpu as pltpu
from jax.experimental.pallas import tpu_sc as plsc

EMBED = 64
WIDE = 128
SEQ = 50
NUM_WORKERS = 32
BATCH = 4096
FLAT = BATCH * SEQ
PER_W = FLAT // NUM_WORKERS
CHUNK = 200
GATHER = 100
NCHUNK = PER_W // CHUNK
NBUF = 4
LANES = 16


def _build_kernel():
    mesh = plsc.VectorSubcoreMesh(core_axis_name="c", subcore_axis_name="s")

    @functools.partial(
        pl.kernel,
        mesh=mesh,
        out_type=jax.ShapeDtypeStruct((FLAT, WIDE), jnp.float32),
        scratch_types=[
            pltpu.VMEM((PER_W // GATHER, GATHER), jnp.int32),
            pltpu.VMEM((SEQ, EMBED), jnp.float32),
        ]
        + [pltpu.VMEM((CHUNK, WIDE), jnp.float32) for _ in range(NBUF)]
        + [pltpu.SemaphoreType.DMA for _ in range(2 * NBUF)],
    )
    def k(x_hbm, table_hbm, pe_hbm, out_hbm, idx_v, pe_v, *bufs_and_sems):
        rows = bufs_and_sems[:NBUF]
        sem_g = bufs_and_sems[NBUF:2 * NBUF]
        sem_w = bufs_and_sems[2 * NBUF:]
        cid = lax.axis_index("c")
        sid = lax.axis_index("s")
        wid = sid * 2 + cid
        base_g = wid * (PER_W // GATHER)
        base_o = wid * PER_W
        pltpu.sync_copy(pe_hbm, pe_v)
        pltpu.sync_copy(x_hbm.at[pl.ds(base_g, PER_W // GATHER)], idx_v)

        def start_gather(c, b):
            for g in range(CHUNK // GATHER):
                pltpu.async_copy(
                    table_hbm.at[idx_v.at[c * (CHUNK // GATHER) + g]],
                    rows[b].at[pl.ds(g * GATHER, GATHER)],
                    sem_g[b],
                )

        def wait_gather(c, b):
            for g in range(CHUNK // GATHER):
                pltpu.make_async_copy(
                    table_hbm.at[idx_v.at[c * (CHUNK // GATHER) + g]],
                    rows[b].at[pl.ds(g * GATHER, GATHER)],
                    sem_g[b],
                ).wait()

        def start_write(c, b):
            pltpu.async_copy(
                rows[b],
                out_hbm.at[pl.ds(base_o + c * CHUNK, CHUNK)],
                sem_w[b],
            )

        def wait_write(c, b):
            pltpu.make_async_copy(
                rows[b],
                out_hbm.at[pl.ds(base_o + c * CHUNK, CHUNK)],
                sem_w[b],
            ).wait()

        for b in range(NBUF):
            start_gather(b, b)

        def it_body(it, carry):
            for b in range(NBUF):
                c = it * NBUF + b
                wait_gather(c, b)

                def add_t(t, inner):
                    for j in range(EMBED // LANES):
                        sl = pl.ds(j * LANES, LANES)
                        pe_vec = pe_v[t, sl]
                        for e in range(CHUNK // SEQ):
                            plsc.addupdate(rows[b].at[e * SEQ + t, sl], pe_vec)
                    return inner

                lax.fori_loop(0, SEQ, add_t, 0)
                start_write(c, b)

                @pl.when(it < NCHUNK // NBUF - 1)
                def _():
                    wait_write(c, b)
                    start_gather(c + NBUF, b)

                @pl.when(it == NCHUNK // NBUF - 1)
                def _():
                    wait_write(c, b)

            return carry

        lax.fori_loop(0, NCHUNK // NBUF, it_body, 0)

    return k


_sc_kernel = _build_kernel()


@jax.jit
def kernel(x, table, pe):
    x2 = x.reshape(FLAT // GATHER, GATHER).astype(jnp.int32)
    tab128 = jnp.pad(table, ((0, 0), (0, WIDE - EMBED)))
    out = _sc_kernel(x2, tab128, pe)
    return out[:, :EMBED].reshape(BATCH, SEQ, EMBED)


# speedup vs baseline: 1.3796x; 1.1525x over previous
import functools

import jax
import jax.numpy as jnp
from jax import lax
from jax.experimental import pallas as pl
from jax.experimental.pallas import tpu as pltpu
from jax.experimental.pallas import tpu_sc as plsc

EMBED = 64
WIDE = 128
SEQ = 50
NUM_WORKERS = 32
BATCH = 4096
FLAT = BATCH * SEQ
PER_W = FLAT // NUM_WORKERS
CHUNK = 200
GATHER = 100
PACKED = CHUNK // 2
NCHUNK = PER_W // CHUNK
NBUF = 4
LANES = 16


def _build_kernel():
    mesh = plsc.VectorSubcoreMesh(core_axis_name="c", subcore_axis_name="s")

    @functools.partial(
        pl.kernel,
        mesh=mesh,
        out_type=jax.ShapeDtypeStruct((BATCH, SEQ, WIDE), jnp.float32),
        scratch_types=[
            pltpu.VMEM((PER_W // GATHER, GATHER), jnp.int32),
            pltpu.VMEM((SEQ, EMBED), jnp.float32),
        ]
        + [pltpu.VMEM((CHUNK, WIDE), jnp.float32) for _ in range(NBUF)]
        + [pltpu.SemaphoreType.DMA for _ in range(2 * NBUF)],
    )
    def k(x_hbm, table_hbm, pe_hbm, out_hbm, idx_v, pe_v, *rest):
        gbuf = rest[:NBUF]
        sem_g = rest[NBUF:2 * NBUF]
        sem_w = rest[2 * NBUF:]
        cid = lax.axis_index("c")
        sid = lax.axis_index("s")
        wid = sid * 2 + cid
        base_g = wid * (PER_W // GATHER)
        base_e = wid * (PER_W // SEQ)
        pltpu.sync_copy(pe_hbm, pe_v)
        pltpu.sync_copy(x_hbm.at[pl.ds(base_g, PER_W // GATHER)], idx_v)

        def start_gather(c, b):
            for g in range(CHUNK // GATHER):
                pltpu.async_copy(
                    table_hbm.at[idx_v.at[c * (CHUNK // GATHER) + g]],
                    gbuf[b].at[pl.ds(g * GATHER, GATHER)],
                    sem_g[b],
                )

        def wait_gather(c, b):
            for g in range(CHUNK // GATHER):
                pltpu.make_async_copy(
                    table_hbm.at[idx_v.at[c * (CHUNK // GATHER) + g]],
                    gbuf[b].at[pl.ds(g * GATHER, GATHER)],
                    sem_g[b],
                ).wait()

        def start_write(c, b):
            for e in range(CHUNK // SEQ):
                pltpu.async_copy(
                    gbuf[b].at[pl.ds(e * SEQ, SEQ)],
                    out_hbm.at[base_e + c * (CHUNK // SEQ) + e],
                    sem_w[b],
                )

        def wait_write(c, b):
            for e in range(CHUNK // SEQ):
                pltpu.make_async_copy(
                    gbuf[b].at[pl.ds(e * SEQ, SEQ)],
                    out_hbm.at[base_e + c * (CHUNK // SEQ) + e],
                    sem_w[b],
                ).wait()

        for b in range(NBUF):
            start_gather(b, b)

        def it_body(it, carry):
            for b in range(NBUF):
                c = it * NBUF + b
                wait_gather(c, b)

                def add_t(t, inner):
                    for j in range(EMBED // LANES):
                        sl = pl.ds(j * LANES, LANES)
                        pe_vec = pe_v[t, sl]
                        for e in range(CHUNK // SEQ):
                            plsc.addupdate(gbuf[b].at[e * SEQ + t, sl], pe_vec)
                    return inner

                lax.fori_loop(0, SEQ, add_t, 0)
                start_write(c, b)
                wait_write(c, b)

                @pl.when(it < NCHUNK // NBUF - 1)
                def _():
                    start_gather(c + NBUF, b)

            return carry

        lax.fori_loop(0, NCHUNK // NBUF, it_body, 0)

    return k


_sc_kernel = _build_kernel()


@jax.jit
def kernel(x, table, pe):
    x2 = x.reshape(FLAT // GATHER, GATHER).astype(jnp.int32)
    tab128 = jnp.pad(table, ((0, 0), (0, WIDE - EMBED)))
    out = _sc_kernel(x2, tab128, pe)
    return out[:, :, :EMBED]
